# baseline (speedup 1.0000x reference)
import functools

import jax
import jax.numpy as jnp
from jax import lax
from jax.experimental import pallas as pl
from jax.experimental.pallas import tpu as pltpu
from jax.experimental.pallas import tpu_sc as plsc

NC, NS = 2, 16
NW = NC * NS
NSLOT = 3
GS = 8
BS = 512


def _build_gather(B, S, D):
    assert S % NW == 0
    s_per_w = S // NW
    assert s_per_w % GS == 0
    n_chunks = s_per_w // GS
    ROWS = B * GS

    mesh = plsc.VectorSubcoreMesh(
        core_axis_name="c", subcore_axis_name="s", num_cores=NC, num_subcores=NS
    )

    @functools.partial(
        pl.kernel,
        out_type=jax.ShapeDtypeStruct((B, S, D), jnp.float32),
        mesh=mesh,
        scratch_types=[
            pltpu.VMEM((B, s_per_w), jnp.int32),
            pltpu.VMEM((NSLOT, ROWS, D), jnp.float32),
            pltpu.SemaphoreType.DMA((NSLOT,)),
            pltpu.SemaphoreType.DMA((NSLOT,)),
        ],
    )
    def gather(ids_hbm, tok_hbm, out_hbm, idx_all, buf, isem, osem):
        wid = lax.axis_index("s") * NC + lax.axis_index("c")
        s0 = wid * s_per_w

        for b in range(B):
            pltpu.sync_copy(ids_hbm.at[b, pl.ds(s0, s_per_w)], idx_all.at[b])

        def in_copies(c, s):
            sb = c * GS
            return [
                pltpu.make_async_copy(
                    tok_hbm.at[idx_all.at[b, pl.ds(sb, GS)]],
                    buf.at[s, pl.ds(b * GS, GS)],
                    isem.at[s],
                )
                for b in range(B)
            ]

        def out_copies(c, s):
            sb = c * GS
            return [
                pltpu.make_async_copy(
                    buf.at[s, pl.ds(b * GS, GS)],
                    out_hbm.at[b, pl.ds(s0 + sb, GS)],
                    osem.at[s],
                )
                for b in range(B)
            ]

        def fire(cps):
            for cp in cps:
                cp.start()

        def drain(cps):
            for cp in cps:
                cp.wait()

        fire(in_copies(0, 0))
        fire(in_copies(1, 1))

        def chunk_body(c, carry):
            s = lax.rem(c, NSLOT)
            drain(in_copies(c, s))
            fire(out_copies(c, s))

            @pl.when(c + 2 < n_chunks)
            def _refill():
                s2 = lax.rem(c + 2, NSLOT)

                @pl.when(c >= 1)
                def _drain_prev_out():
                    drain(out_copies(c - 1, s2))

                fire(in_copies(c + 2, s2))

            return carry

        lax.fori_loop(0, n_chunks, chunk_body, 0)
        drain(out_copies(n_chunks - 2, (n_chunks - 2) % NSLOT))
        drain(out_copies(n_chunks - 1, (n_chunks - 1) % NSLOT))

    return gather


def _build_ln(B, S, D):
    assert S % BS == 0

    def body(tok_ref, pos_ref, gam_ref, bet_ref, o_ref):
        x = tok_ref[0] + pos_ref[...]
        mean = jnp.mean(x, axis=-1, keepdims=True)
        var = jnp.mean(x * x, axis=-1, keepdims=True) - mean * mean
        rstd = lax.rsqrt(var + 1e-5)
        o_ref[0] = (x * rstd - mean * rstd) * gam_ref[0] + bet_ref[0]

    return pl.pallas_call(
        body,
        grid=(S // BS, B),
        in_specs=[
            pl.BlockSpec((1, BS, D), lambda i, j: (j, i, 0)),
            pl.BlockSpec((BS, D), lambda i, j: (i, 0)),
            pl.BlockSpec((1, D), lambda i, j: (0, 0)),
            pl.BlockSpec((1, D), lambda i, j: (0, 0)),
        ],
        out_specs=pl.BlockSpec((1, BS, D), lambda i, j: (j, i, 0)),
        out_shape=jax.ShapeDtypeStruct((B, S, D), jnp.float32),
    )


def kernel(input_ids, tok_table, pos_table, ln_gamma, ln_beta):
    B, S = input_ids.shape
    _, D = tok_table.shape
    gathered = _build_gather(B, S, D)(
        input_ids.astype(jnp.int32), tok_table
    )
    return _build_ln(B, S, D)(
        gathered,
        pos_table[:S],
        ln_gamma.reshape(1, D),
        ln_beta.reshape(1, D),
    )


# speedup vs baseline: 1.0176x; 1.0176x over previous
import functools

import jax
import jax.numpy as jnp
from jax import lax
from jax.experimental import pallas as pl
from jax.experimental.pallas import tpu as pltpu
from jax.experimental.pallas import tpu_sc as plsc

NC, NS = 2, 16
NW = NC * NS
NSLOT = 3
GS = 8
BS = 512
K = 2


def _build_gather(B, S, D):
    assert S % NW == 0
    s_per_w = S // NW
    assert s_per_w % GS == 0
    n_chunks = s_per_w // GS
    ROWS = B * GS

    mesh = plsc.VectorSubcoreMesh(
        core_axis_name="c", subcore_axis_name="s", num_cores=NC, num_subcores=NS
    )

    @functools.partial(
        pl.kernel,
        out_type=jax.ShapeDtypeStruct((B, S, D), jnp.float32),
        mesh=mesh,
        scratch_types=[
            pltpu.VMEM((B, s_per_w), jnp.int32),
            pltpu.VMEM((NSLOT, ROWS, D), jnp.float32),
            pltpu.SemaphoreType.DMA((NSLOT,)),
            pltpu.SemaphoreType.DMA((NSLOT,)),
        ],
    )
    def gather(ids_hbm, tok_hbm, out_hbm, idx_all, buf, isem, osem):
        wid = lax.axis_index("s") * NC + lax.axis_index("c")
        s0 = wid * s_per_w

        for b in range(B):
            pltpu.sync_copy(ids_hbm.at[b, pl.ds(s0, s_per_w)], idx_all.at[b])

        def in_copies(c, s):
            sb = c * GS
            return [
                pltpu.make_async_copy(
                    tok_hbm.at[idx_all.at[b, pl.ds(sb, GS)]],
                    buf.at[s, pl.ds(b * GS, GS)],
                    isem.at[s],
                )
                for b in range(B)
            ]

        def out_copies(c, s):
            sb = c * GS
            return [
                pltpu.make_async_copy(
                    buf.at[s, pl.ds(b * GS, GS)],
                    out_hbm.at[b, pl.ds(s0 + sb, GS)],
                    osem.at[s],
                )
                for b in range(B)
            ]

        def fire(cps):
            for cp in cps:
                cp.start()

        def drain(cps):
            for cp in cps:
                cp.wait()

        fire(in_copies(0, 0))
        fire(in_copies(1, 1))

        def chunk_body(c, carry):
            s = lax.rem(c, NSLOT)
            drain(in_copies(c, s))
            fire(out_copies(c, s))

            @pl.when(c + 2 < n_chunks)
            def _refill():
                s2 = lax.rem(c + 2, NSLOT)

                @pl.when(c >= 1)
                def _drain_prev_out():
                    drain(out_copies(c - 1, s2))

                fire(in_copies(c + 2, s2))

            return carry

        lax.fori_loop(0, n_chunks, chunk_body, 0)
        for c in range(max(n_chunks - 3, 0), n_chunks):
            drain(out_copies(c, c % NSLOT))

    return gather


def _ln_body(tok_ref, pos_ref, gam_ref, bet_ref, o_ref):
    x = tok_ref[0] + pos_ref[...]
    mean = jnp.mean(x, axis=-1, keepdims=True)
    var = jnp.mean(x * x, axis=-1, keepdims=True) - mean * mean
    rstd = lax.rsqrt(var + 1e-5)
    o_ref[0] = (x * rstd - mean * rstd) * gam_ref[0] + bet_ref[0]


def _build_ln_first(B, S, D, Sk):
    assert Sk % BS == 0

    return pl.pallas_call(
        _ln_body,
        grid=(Sk // BS, B),
        in_specs=[
            pl.BlockSpec((1, BS, D), lambda i, j: (j, i, 0)),
            pl.BlockSpec((BS, D), lambda i, j: (i, 0)),
            pl.BlockSpec((1, D), lambda i, j: (0, 0)),
            pl.BlockSpec((1, D), lambda i, j: (0, 0)),
        ],
        out_specs=pl.BlockSpec((1, BS, D), lambda i, j: (j, i, 0)),
        out_shape=jax.ShapeDtypeStruct((B, S, D), jnp.float32),
    )


def _build_ln_slice(B, S, D, Sk, k):
    off = k * (Sk // BS)

    def body(tok_ref, pos_ref, gam_ref, bet_ref, prev_ref, o_ref):
        del prev_ref
        _ln_body(tok_ref, pos_ref, gam_ref, bet_ref, o_ref)

    return pl.pallas_call(
        body,
        grid=(Sk // BS, B),
        in_specs=[
            pl.BlockSpec((1, BS, D), lambda i, j: (j, i, 0)),
            pl.BlockSpec((BS, D), lambda i, j: (i, 0)),
            pl.BlockSpec((1, D), lambda i, j: (0, 0)),
            pl.BlockSpec((1, D), lambda i, j: (0, 0)),
            pl.BlockSpec(memory_space=pl.ANY),
        ],
        out_specs=pl.BlockSpec((1, BS, D), lambda i, j: (j, off + i, 0)),
        out_shape=jax.ShapeDtypeStruct((B, S, D), jnp.float32),
        input_output_aliases={4: 0},
    )


def kernel(input_ids, tok_table, pos_table, ln_gamma, ln_beta):
    B, S = input_ids.shape
    _, D = tok_table.shape
    Sk = S // K
    ids = input_ids.astype(jnp.int32)
    gam = ln_gamma.reshape(1, D)
    bet = ln_beta.reshape(1, D)

    sc_gather = _build_gather(B, Sk, D)
    gathered = [
        sc_gather(ids[:, k * Sk:(k + 1) * Sk], tok_table) for k in range(K)
    ]

    out = _build_ln_first(B, S, D, Sk)(
        gathered[0], pos_table[:Sk], gam, bet
    )
    for k in range(1, K):
        out = _build_ln_slice(B, S, D, Sk, k)(
            gathered[k], pos_table[k * Sk:(k + 1) * Sk], gam, bet, out
        )
    return out
